# baseline (speedup 1.0000x reference)
import functools

import jax
import jax.numpy as jnp
from jax import lax
from jax.experimental import pallas as pl
from jax.experimental.pallas import tpu as pltpu
from jax.experimental.pallas import tpu_sc as plsc

_D = 32
_W = 128


@functools.lru_cache(maxsize=None)
def _make_gather(n_rows: int, vocab_w: int):
    info = plsc.get_sparse_core_info()
    nc, ns = info.num_cores, info.num_subcores
    nw = nc * ns
    n_wide = n_rows // 4
    assert n_rows % nw == 0
    b_per_w = n_rows // nw
    chunk = 256
    n_chunks = b_per_w // chunk
    nbuf = 2

    mesh = plsc.VectorSubcoreMesh(core_axis_name="c", subcore_axis_name="s")

    @functools.partial(
        pl.kernel,
        mesh=mesh,
        out_type=jax.ShapeDtypeStruct((n_wide, _W), jnp.float32),
        scratch_types=[
            pltpu.VMEM((b_per_w,), jnp.int32),
            pltpu.VMEM((b_per_w,), jnp.int32),
            [pltpu.VMEM((chunk, _W), jnp.float32) for _ in range(nbuf)],
            [pltpu.SemaphoreType.DMA for _ in range(nbuf)],
            [pltpu.SemaphoreType.DMA for _ in range(nbuf)],
        ],
    )
    def gather_kernel(table_hbm, idx_hbm, out_hbm, idx_v, widx_v, rows, gsem, osem):
        wid = lax.axis_index("s") * nc + lax.axis_index("c")
        base = wid * b_per_w
        pltpu.sync_copy(idx_hbm.at[pl.ds(base, b_per_w)], idx_v)

        def widx_body(g, _):
            v = idx_v[pl.ds(g * 16, 16)]
            widx_v[pl.ds(g * 16, 16)] = jax.lax.shift_right_logical(v, 2)
            return 0

        lax.fori_loop(0, b_per_w // 16, widx_body, 0)

        def gather(i, b):
            return pltpu.make_async_copy(
                table_hbm.at[widx_v.at[pl.ds(i * chunk, chunk)]], rows[b], gsem[b]
            )

        def out_copy(i, b):
            return pltpu.make_async_copy(
                rows[b].at[pl.ds(0, chunk // 4)],
                out_hbm.at[
                    pl.ds(pl.multiple_of(base // 4 + i * (chunk // 4), 8), chunk // 4)
                ],
                osem[b],
            )

        for i in range(nbuf):
            gather(i, i).start()
        for i in range(n_chunks):
            b = i % nbuf
            gather(i, b).wait()
            out_copy(i, b).start()
            j = i + nbuf
            if j < n_chunks:
                out_copy(i, b).wait()
                gather(j, b).start()
        for i in range(n_chunks - nbuf, n_chunks):
            out_copy(i, i % nbuf).wait()

    return gather_kernel


def kernel(x, table):
    b, s = x.shape
    n = b * s
    idx_flat = x.reshape(n).astype(jnp.int32)
    table_w = table.reshape(table.shape[0] // 4, _W)
    out_w = _make_gather(n, table_w.shape[0])(table_w, idx_flat)
    return out_w.reshape(b, s, _D)


# speedup vs baseline: 1.0798x; 1.0798x over previous
import functools

import jax
import jax.numpy as jnp
from jax import lax
from jax.experimental import pallas as pl
from jax.experimental.pallas import tpu as pltpu
from jax.experimental.pallas import tpu_sc as plsc

_D = 32


@functools.lru_cache(maxsize=None)
def _make_gather(b_dim: int, s_dim: int, vocab: int):
    info = plsc.get_sparse_core_info()
    nc, ns = info.num_cores, info.num_subcores
    nw = nc * ns
    assert b_dim % nw == 0
    d0_per_w = b_dim // nw
    b_per_w = d0_per_w * s_dim
    c0 = 32
    fpc = c0 * s_dim
    n_chunks = d0_per_w // c0
    nbuf = 3

    mesh = plsc.VectorSubcoreMesh(core_axis_name="c", subcore_axis_name="s")

    @functools.partial(
        pl.kernel,
        mesh=mesh,
        out_type=jax.ShapeDtypeStruct((b_dim, s_dim, _D), jnp.float32),
        scratch_types=[
            pltpu.VMEM((b_per_w,), jnp.int32),
            [pltpu.VMEM((fpc, _D), jnp.float32) for _ in range(nbuf)],
            [pltpu.SemaphoreType.DMA for _ in range(nbuf)],
            [pltpu.SemaphoreType.DMA for _ in range(nbuf)],
        ],
        compiler_params=pltpu.CompilerParams(use_tc_tiling_on_sc=False),
    )
    def gather_kernel(table_hbm, idx_hbm, out_hbm, idx_v, rows, gsem, osem):
        wid = lax.axis_index("s") * nc + lax.axis_index("c")
        base = wid * b_per_w
        d0_base = wid * d0_per_w
        pltpu.sync_copy(idx_hbm.at[pl.ds(base, b_per_w)], idx_v)

        def gather(i, b):
            return pltpu.make_async_copy(
                table_hbm.at[idx_v.at[pl.ds(i * fpc, fpc)]], rows[b], gsem[b]
            )

        def out_dma(i, b, k):
            return pltpu.make_async_copy(
                rows[b].at[pl.ds(k * s_dim, s_dim)],
                out_hbm.at[d0_base + i * c0 + k],
                osem[b],
            )

        for i in range(nbuf):
            gather(i, i).start()
        for i in range(n_chunks):
            b = i % nbuf
            gather(i, b).wait()
            for k in range(c0):
                out_dma(i, b, k).start()
            j = i + nbuf
            if j < n_chunks:
                for k in range(c0):
                    out_dma(i, b, k).wait()
                gather(j, b).start()
        for i in range(n_chunks - nbuf, n_chunks):
            for k in range(c0):
                out_dma(i, i % nbuf, k).wait()

    return gather_kernel


def kernel(x, table):
    b, s = x.shape
    idx_flat = x.reshape(b * s).astype(jnp.int32)
    return _make_gather(b, s, table.shape[0])(table, idx_flat)
